# Initial kernel scaffold; baseline (speedup 1.0000x reference)
import jax
import jax.numpy as jnp
from jax.experimental import pallas as pl


def kernel(node_feat, edge_index, edge_feat, triplets, emb, W_rel_0, W_self_0, b_0, W_rel_1, W_self_1, b_1, W_mlp, b_mlp):
    raise NotImplementedError("write your pallas kernel here")


# speedup vs baseline: 2.1828x; 2.1828x over previous
import functools

import jax
import jax.numpy as jnp
from jax import lax
from jax.experimental import pallas as pl
from jax.experimental.pallas import tpu as pltpu
from jax.experimental.pallas import tpu_sc as plsc

f32 = jnp.float32
i32 = jnp.int32

NC, NS, LANES = 2, 16, 16
NW = NC * NS


def _mesh():
    return plsc.VectorSubcoreMesh(core_axis_name="c", subcore_axis_name="s",
                                  num_cores=NC, num_subcores=NS)


def _wid():
    return lax.axis_index("s") * NC + lax.axis_index("c")


def _emb_gather(emb, nfp, NP, D):
    npw = NP // NW
    CH = 64
    nch = npw // CH

    def body(emb_hbm, idx_hbm, out_hbm, idx_v, rows_v, sem):
        wid = _wid()
        for k in range(nch):
            base = wid * npw + k * CH
            pltpu.sync_copy(idx_hbm.at[pl.ds(base, CH)], idx_v)
            pltpu.async_copy(emb_hbm.at[idx_v], rows_v, sem).wait()
            pltpu.sync_copy(rows_v, out_hbm.at[pl.ds(base, CH)])

    return pl.kernel(
        body,
        out_type=jax.ShapeDtypeStruct((NP, D), f32),
        mesh=_mesh(),
        scratch_types=[
            pltpu.VMEM((CH,), i32),
            pltpu.VMEM((CH, D), f32),
            pltpu.SemaphoreType.DMA,
        ],
    )(emb, nfp)


def _edge_agg(xflat, fidx2d, didx2d, N, D, nch_e, stripe):
    acc_rows = NS * stripe

    def body(x_hbm, fidx_hbm, didx_hbm, out_hbm, fbuf, dbuf, rows_v, acc, sem):
        c = lax.axis_index("c")
        s = lax.axis_index("s")
        wid = s * NC + c
        pltpu.sync_copy(fidx_hbm.at[pl.ds(wid * nch_e, nch_e)], fbuf)
        pltpu.sync_copy(didx_hbm.at[pl.ds(wid * nch_e, nch_e)], dbuf)

        z = jnp.zeros((LANES,), f32)

        @pl.loop(0, 128)
        def _(i):
            for j in range(D // LANES):
                rows_v[i, pl.ds(j * LANES, LANES)] = z

        for k in range(stripe // 128):
            pltpu.sync_copy(rows_v, acc.at[pl.ds(s * stripe + k * 128, 128)])
        plsc.subcore_barrier()

        @pl.loop(0, nch_e)
        def _(ch):
            pltpu.async_copy(x_hbm.at[fbuf.at[ch]], rows_v, sem).wait()
            pltpu.sync_copy(rows_v, acc.at[dbuf.at[ch]], add=True)

        plsc.subcore_barrier()
        for k in range(stripe // 128):
            r0 = s * stripe + k * 128
            pltpu.sync_copy(acc.at[pl.ds(r0, 128)], rows_v)
            pltpu.sync_copy(rows_v, out_hbm.at[c, pl.ds(r0, 128)])

    return pl.kernel(
        body,
        out_type=jax.ShapeDtypeStruct((NC, acc_rows, D), f32),
        mesh=_mesh(),
        scratch_types=[
            pltpu.VMEM((nch_e, 128), i32),
            pltpu.VMEM((nch_e, 128), i32),
            pltpu.VMEM((128, D), f32),
            pltpu.VMEM_SHARED((acc_rows, D), f32),
            pltpu.SemaphoreType.DMA,
        ],
    )(xflat, fidx2d, didx2d)


def _trip_score(P, Q, t0, t2, T, OUT):
    tpw = T // NW
    CH = 80
    nch = tpw // CH

    def body(p_hbm, q_hbm, t0_hbm, t2_hbm, out_hbm,
             t0buf, t2buf, pbuf, qbuf, obuf, sem, sem2):
        wid = _wid()
        pltpu.sync_copy(t0_hbm.at[pl.ds(wid * tpw, tpw)], t0buf)
        pltpu.sync_copy(t2_hbm.at[pl.ds(wid * tpw, tpw)], t2buf)

        @pl.loop(0, nch)
        def _(ch):
            a = pltpu.async_copy(p_hbm.at[t0buf.at[pl.ds(ch * CH, CH)]],
                                 pbuf, sem)
            b = pltpu.async_copy(q_hbm.at[t2buf.at[pl.ds(ch * CH, CH)]],
                                 qbuf, sem2)
            a.wait()
            b.wait()
            for i in range(CH):
                obuf[i, :] = pbuf[i, :] + qbuf[i, :]
            pltpu.sync_copy(obuf, out_hbm.at[pl.ds(wid * tpw + ch * CH, CH)])

    return pl.kernel(
        body,
        out_type=jax.ShapeDtypeStruct((T, OUT), f32),
        mesh=_mesh(),
        compiler_params=pltpu.CompilerParams(use_tc_tiling_on_sc=False),
        scratch_types=[
            pltpu.VMEM((tpw,), i32),
            pltpu.VMEM((tpw,), i32),
            pltpu.VMEM((CH, OUT), f32),
            pltpu.VMEM((CH, OUT), f32),
            pltpu.VMEM((CH, OUT), f32),
            pltpu.SemaphoreType.DMA,
            pltpu.SemaphoreType.DMA,
        ],
    )(P, Q, t0, t2)


_BLK = 2000


def _xmat(h, Wcat, N, D):
    R1 = Wcat.shape[0]
    nb = N // _BLK

    def body(h_ref, w_ref, o_ref):
        o_ref[0] = jnp.dot(h_ref[...], w_ref[0], preferred_element_type=f32)

    return pl.pallas_call(
        body,
        grid=(nb, R1),
        in_specs=[
            pl.BlockSpec((_BLK, D), lambda j, r: (j, 0)),
            pl.BlockSpec((1, D, D), lambda j, r: (r, 0, 0)),
        ],
        out_specs=pl.BlockSpec((1, _BLK, D), lambda j, r: (r, j, 0)),
        out_shape=jax.ShapeDtypeStruct((R1, N, D), f32),
    )(h, Wcat)


def _combine_xmat(agg, Xprev, b, Wcat, N, D):
    R1 = Wcat.shape[0]
    R = R1 - 1
    nb = N // _BLK

    def body(agg_ref, sx_ref, b_ref, w_ref, o_ref, h_scr):
        @pl.when(pl.program_id(1) == 0)
        def _():
            h_scr[...] = (agg_ref[0] + agg_ref[1] + sx_ref[0] + b_ref[0])
        o_ref[0] = jnp.dot(h_scr[...], w_ref[0], preferred_element_type=f32)

    return pl.pallas_call(
        body,
        grid=(nb, R1),
        in_specs=[
            pl.BlockSpec((NC, _BLK, D), lambda j, r: (0, j, 0)),
            pl.BlockSpec((1, _BLK, D), lambda j, r: (R, j, 0)),
            pl.BlockSpec((1, D), lambda j, r: (0, 0)),
            pl.BlockSpec((1, D, D), lambda j, r: (r, 0, 0)),
        ],
        out_specs=pl.BlockSpec((1, _BLK, D), lambda j, r: (r, j, 0)),
        out_shape=jax.ShapeDtypeStruct((R1, N, D), f32),
        scratch_shapes=[pltpu.VMEM((_BLK, D), f32)],
    )(agg, Xprev, b, Wcat)


def _final_pq(agg, Xprev, b, Wt, Wb, bm, N, D, OUT):
    R = Xprev.shape[0] - 1
    nb = N // _BLK

    def body(agg_ref, sx_ref, b_ref, wt_ref, wb_ref, bm_ref, p_ref, q_ref):
        h2 = agg_ref[0] + agg_ref[1] + sx_ref[0] + b_ref[0]
        p_ref[...] = jnp.dot(h2, wt_ref[...], preferred_element_type=f32) \
            + bm_ref[0]
        q_ref[...] = jnp.dot(h2, wb_ref[...], preferred_element_type=f32)

    return pl.pallas_call(
        body,
        grid=(nb,),
        in_specs=[
            pl.BlockSpec((NC, _BLK, D), lambda j: (0, j, 0)),
            pl.BlockSpec((1, _BLK, D), lambda j: (R, j, 0)),
            pl.BlockSpec((1, D), lambda j: (0, 0)),
            pl.BlockSpec((D, OUT), lambda j: (0, 0)),
            pl.BlockSpec((D, OUT), lambda j: (0, 0)),
            pl.BlockSpec((1, OUT), lambda j: (0, 0)),
        ],
        out_specs=[
            pl.BlockSpec((_BLK, OUT), lambda j: (j, 0)),
            pl.BlockSpec((_BLK, OUT), lambda j: (j, 0)),
        ],
        out_shape=[
            jax.ShapeDtypeStruct((N, OUT), f32),
            jax.ShapeDtypeStruct((N, OUT), f32),
        ],
    )(agg, Xprev, b, Wt, Wb, bm)


def kernel(node_feat, edge_index, edge_feat, triplets, emb,
           W_rel_0, W_self_0, b_0, W_rel_1, W_self_1, b_1, W_mlp, b_mlp):
    N, D = emb.shape
    R = W_rel_0.shape[0]
    E = edge_feat.shape[0]
    T = triplets.shape[0]
    OUT = W_mlp.shape[1]

    src = edge_index[0].astype(i32)
    dst = edge_index[1].astype(i32)
    et = edge_feat.astype(i32)
    flat = et * N + src

    nch_e = 8 * (-(-E // (NW * 128 * 8)))
    EP = NW * nch_e * 128
    flatp = jnp.concatenate([flat, jnp.zeros((EP - E,), i32)])
    dstp = jnp.concatenate([dst, jnp.full((EP - E,), N, i32)])
    fidx2d = flatp.reshape(NW * nch_e, 128)
    didx2d = dstp.reshape(NW * nch_e, 128)
    stripe = 128 * (-(-(N + 1) // (NS * 128)))

    NP = NW * 64 * (-(-N // (NW * 64)))
    nfp = jnp.concatenate([node_feat.astype(i32),
                           jnp.zeros((NP - N,), i32)])

    t0 = triplets[:, 0].astype(i32)
    t2 = triplets[:, 2].astype(i32)

    Wcat0 = jnp.concatenate([W_rel_0, W_self_0[None]], axis=0)
    Wcat1 = jnp.concatenate([W_rel_1, W_self_1[None]], axis=0)
    b0 = b_0.reshape(1, D)
    b1 = b_1.reshape(1, D)
    bm = b_mlp.reshape(1, OUT)
    Wt = W_mlp[:D]
    Wb = W_mlp[D:]

    h0 = _emb_gather(emb, nfp, NP, D)
    X0 = _xmat(h0, Wcat0, N, D)
    agg0 = _edge_agg(X0.reshape((R + 1) * N, D),
                     fidx2d, didx2d, N, D, nch_e, stripe)
    X1 = _combine_xmat(agg0, X0, b0, Wcat1, N, D)
    agg1 = _edge_agg(X1.reshape((R + 1) * N, D),
                     fidx2d, didx2d, N, D, nch_e, stripe)
    P, Q = _final_pq(agg1, X1, b1, Wt, Wb, bm, N, D, OUT)
    score = _trip_score(P, Q, t0, t2, T, OUT)
    return score


# speedup vs baseline: 2.4748x; 1.1338x over previous
import functools

import jax
import jax.numpy as jnp
from jax import lax
from jax.experimental import pallas as pl
from jax.experimental.pallas import tpu as pltpu
from jax.experimental.pallas import tpu_sc as plsc

f32 = jnp.float32
i32 = jnp.int32

NC, NS, LANES = 2, 16, 16
NW = NC * NS


def _mesh():
    return plsc.VectorSubcoreMesh(core_axis_name="c", subcore_axis_name="s",
                                  num_cores=NC, num_subcores=NS)


def _wid():
    return lax.axis_index("s") * NC + lax.axis_index("c")


def _emb_gather(emb, nfp, NP, D):
    npw = NP // NW
    CH = 64
    nch = npw // CH

    def body(emb_hbm, idx_hbm, out_hbm, idx_v, rows_v, sem):
        wid = _wid()
        for k in range(nch):
            base = wid * npw + k * CH
            pltpu.sync_copy(idx_hbm.at[pl.ds(base, CH)], idx_v)
            pltpu.async_copy(emb_hbm.at[idx_v], rows_v, sem).wait()
            pltpu.sync_copy(rows_v, out_hbm.at[pl.ds(base, CH)])

    return pl.kernel(
        body,
        out_type=jax.ShapeDtypeStruct((NP, D), f32),
        mesh=_mesh(),
        scratch_types=[
            pltpu.VMEM((CH,), i32),
            pltpu.VMEM((CH, D), f32),
            pltpu.SemaphoreType.DMA,
        ],
    )(emb, nfp)


def _edge_agg(xflat, fd3d, N, D, nch_e, stripe):
    acc_rows = NS * stripe
    GRP = 4
    ngr = nch_e // GRP

    def body(x_hbm, fd_hbm, out_hbm, ib0, ib1, r0, r1, acc, g0, g1):
        ibs = (ib0, ib1)
        rbufs = (r0, r1)
        gsems = (g0, g1)
        c = lax.axis_index("c")
        s = lax.axis_index("s")
        wid = s * NC + c
        gbase = wid * ngr

        z = jnp.zeros((LANES,), f32)

        @pl.loop(0, 128)
        def _(i):
            for j in range(D // LANES):
                r0[i, pl.ds(j * LANES, LANES)] = z

        for k in range(stripe // 128):
            pltpu.sync_copy(r0, acc.at[pl.ds(s * stripe + k * 128, 128)])
        plsc.subcore_barrier()

        def issue(idx_row, b):
            pltpu.async_copy(x_hbm.at[idx_row], rbufs[b], gsems[b])

        def wait_gather(b):
            pltpu.make_async_copy(x_hbm.at[pl.ds(0, 128)], rbufs[b],
                                  gsems[b]).wait()

        def do_group(g, par, load_next, last):
            ib, ibn = ibs[par], ibs[1 - par]
            if load_next:
                pltpu.sync_copy(fd_hbm.at[gbase + g + 1], ibn)
            for j in range(GRP):
                b = j % 2
                wait_gather(b)
                pltpu.sync_copy(rbufs[b], acc.at[ib.at[GRP + j]], add=True)
                if j < 2:
                    issue(ib.at[j + 2], b)
                elif not last:
                    issue(ibn.at[j - 2], b)

        pltpu.sync_copy(fd_hbm.at[gbase], ib0)
        issue(ib0.at[0], 0)
        issue(ib0.at[1], 1)

        @pl.loop(0, ngr // 2 - 1)
        def _(t):
            do_group(2 * t, 0, True, False)
            do_group(2 * t + 1, 1, True, False)

        do_group(ngr - 2, 0, True, False)
        do_group(ngr - 1, 1, False, True)

        plsc.subcore_barrier()
        for k in range(stripe // 128):
            rr = s * stripe + k * 128
            pltpu.sync_copy(acc.at[pl.ds(rr, 128)], r0)
            pltpu.sync_copy(r0, out_hbm.at[c, pl.ds(rr, 128)])

    return pl.kernel(
        body,
        out_type=jax.ShapeDtypeStruct((NC, acc_rows, D), f32),
        mesh=_mesh(),
        scratch_types=[
            pltpu.VMEM((8, 128), i32),
            pltpu.VMEM((8, 128), i32),
            pltpu.VMEM((128, D), f32),
            pltpu.VMEM((128, D), f32),
            pltpu.VMEM_SHARED((acc_rows, D), f32),
            pltpu.SemaphoreType.DMA,
            pltpu.SemaphoreType.DMA,
        ],
    )(xflat, fd3d)


def _trip_score(P, Q, t0, t2, T, OUT):
    tpw = T // NW
    CH = 80
    nch = tpw // CH

    assert nch % 2 == 1

    def body(p_hbm, q_hbm, t0_hbm, t2_hbm, out_hbm, t0buf, t2buf,
             pb0, qb0, ob0, pb1, qb1, ob1, sp0, sq0, sp1, sq1):
        wid = _wid()
        pltpu.sync_copy(t0_hbm.at[pl.ds(wid * tpw, tpw)], t0buf)
        pltpu.sync_copy(t2_hbm.at[pl.ds(wid * tpw, tpw)], t2buf)
        sets = ((pb0, qb0, ob0, sp0, sq0), (pb1, qb1, ob1, sp1, sq1))

        def issue(ch, st):
            pb, qb, _, sp, sq = st
            pltpu.async_copy(p_hbm.at[t0buf.at[pl.ds(ch * CH, CH)]], pb, sp)
            pltpu.async_copy(q_hbm.at[t2buf.at[pl.ds(ch * CH, CH)]], qb, sq)

        def process(ch, st):
            pb, qb, ob, sp, sq = st
            pltpu.make_async_copy(p_hbm.at[pl.ds(0, CH)], pb, sp).wait()
            pltpu.make_async_copy(q_hbm.at[pl.ds(0, CH)], qb, sq).wait()
            for i in range(CH):
                ob[i, :] = pb[i, :] + qb[i, :]
            pltpu.sync_copy(ob, out_hbm.at[pl.ds(wid * tpw + ch * CH, CH)])

        issue(0, sets[0])

        @pl.loop(0, nch // 2)
        def _(g):
            issue(2 * g + 1, sets[1])
            process(2 * g, sets[0])
            issue(2 * g + 2, sets[0])
            process(2 * g + 1, sets[1])

        process(nch - 1, sets[0])

    return pl.kernel(
        body,
        out_type=jax.ShapeDtypeStruct((T, OUT), f32),
        mesh=_mesh(),
        compiler_params=pltpu.CompilerParams(use_tc_tiling_on_sc=False),
        scratch_types=[
            pltpu.VMEM((tpw,), i32),
            pltpu.VMEM((tpw,), i32),
            pltpu.VMEM((CH, OUT), f32),
            pltpu.VMEM((CH, OUT), f32),
            pltpu.VMEM((CH, OUT), f32),
            pltpu.VMEM((CH, OUT), f32),
            pltpu.VMEM((CH, OUT), f32),
            pltpu.VMEM((CH, OUT), f32),
            pltpu.SemaphoreType.DMA,
            pltpu.SemaphoreType.DMA,
            pltpu.SemaphoreType.DMA,
            pltpu.SemaphoreType.DMA,
        ],
    )(P, Q, t0, t2)


_BLK = 2000


def _xmat(h, Wcat, N, D):
    R1 = Wcat.shape[0]
    nb = N // _BLK

    def body(h_ref, w_ref, o_ref):
        o_ref[0] = jnp.dot(h_ref[...], w_ref[0], preferred_element_type=f32)

    return pl.pallas_call(
        body,
        grid=(nb, R1),
        in_specs=[
            pl.BlockSpec((_BLK, D), lambda j, r: (j, 0)),
            pl.BlockSpec((1, D, D), lambda j, r: (r, 0, 0)),
        ],
        out_specs=pl.BlockSpec((1, _BLK, D), lambda j, r: (r, j, 0)),
        out_shape=jax.ShapeDtypeStruct((R1, N, D), f32),
    )(h, Wcat)


def _combine_xmat(agg, Xprev, b, Wcat, N, D):
    R1 = Wcat.shape[0]
    R = R1 - 1
    nb = N // _BLK

    def body(agg_ref, sx_ref, b_ref, w_ref, o_ref, h_scr):
        @pl.when(pl.program_id(1) == 0)
        def _():
            h_scr[...] = (agg_ref[0] + agg_ref[1] + sx_ref[0] + b_ref[0])
        o_ref[0] = jnp.dot(h_scr[...], w_ref[0], preferred_element_type=f32)

    return pl.pallas_call(
        body,
        grid=(nb, R1),
        in_specs=[
            pl.BlockSpec((NC, _BLK, D), lambda j, r: (0, j, 0)),
            pl.BlockSpec((1, _BLK, D), lambda j, r: (R, j, 0)),
            pl.BlockSpec((1, D), lambda j, r: (0, 0)),
            pl.BlockSpec((1, D, D), lambda j, r: (r, 0, 0)),
        ],
        out_specs=pl.BlockSpec((1, _BLK, D), lambda j, r: (r, j, 0)),
        out_shape=jax.ShapeDtypeStruct((R1, N, D), f32),
        scratch_shapes=[pltpu.VMEM((_BLK, D), f32)],
    )(agg, Xprev, b, Wcat)


def _final_pq(agg, Xprev, b, Wt, Wb, bm, N, D, OUT):
    R = Xprev.shape[0] - 1
    nb = N // _BLK

    def body(agg_ref, sx_ref, b_ref, wt_ref, wb_ref, bm_ref, p_ref, q_ref):
        h2 = agg_ref[0] + agg_ref[1] + sx_ref[0] + b_ref[0]
        p_ref[...] = jnp.dot(h2, wt_ref[...], preferred_element_type=f32) \
            + bm_ref[0]
        q_ref[...] = jnp.dot(h2, wb_ref[...], preferred_element_type=f32)

    return pl.pallas_call(
        body,
        grid=(nb,),
        in_specs=[
            pl.BlockSpec((NC, _BLK, D), lambda j: (0, j, 0)),
            pl.BlockSpec((1, _BLK, D), lambda j: (R, j, 0)),
            pl.BlockSpec((1, D), lambda j: (0, 0)),
            pl.BlockSpec((D, OUT), lambda j: (0, 0)),
            pl.BlockSpec((D, OUT), lambda j: (0, 0)),
            pl.BlockSpec((1, OUT), lambda j: (0, 0)),
        ],
        out_specs=[
            pl.BlockSpec((_BLK, OUT), lambda j: (j, 0)),
            pl.BlockSpec((_BLK, OUT), lambda j: (j, 0)),
        ],
        out_shape=[
            jax.ShapeDtypeStruct((N, OUT), f32),
            jax.ShapeDtypeStruct((N, OUT), f32),
        ],
    )(agg, Xprev, b, Wt, Wb, bm)


def kernel(node_feat, edge_index, edge_feat, triplets, emb,
           W_rel_0, W_self_0, b_0, W_rel_1, W_self_1, b_1, W_mlp, b_mlp):
    N, D = emb.shape
    R = W_rel_0.shape[0]
    E = edge_feat.shape[0]
    T = triplets.shape[0]
    OUT = W_mlp.shape[1]

    src = edge_index[0].astype(i32)
    dst = edge_index[1].astype(i32)
    et = edge_feat.astype(i32)
    flat = et * N + src

    nch_e = 8 * (-(-E // (NW * 128 * 8)))
    EP = NW * nch_e * 128
    ngr = nch_e // 4
    flatp = jnp.concatenate([flat, jnp.zeros((EP - E,), i32)])
    dstp = jnp.concatenate([dst, jnp.full((EP - E,), N, i32)])
    fd3d = jnp.concatenate(
        [flatp.reshape(NW, ngr, 4, 128), dstp.reshape(NW, ngr, 4, 128)],
        axis=2).reshape(NW * ngr, 8, 128)
    stripe = 128 * (-(-(N + 1) // (NS * 128)))

    NP = NW * 64 * (-(-N // (NW * 64)))
    nfp = jnp.concatenate([node_feat.astype(i32),
                           jnp.zeros((NP - N,), i32)])

    t0 = triplets[:, 0].astype(i32)
    t2 = triplets[:, 2].astype(i32)

    Wcat0 = jnp.concatenate([W_rel_0, W_self_0[None]], axis=0)
    Wcat1 = jnp.concatenate([W_rel_1, W_self_1[None]], axis=0)
    b0 = b_0.reshape(1, D)
    b1 = b_1.reshape(1, D)
    bm = b_mlp.reshape(1, OUT)
    Wt = W_mlp[:D]
    Wb = W_mlp[D:]

    h0 = _emb_gather(emb, nfp, NP, D)
    X0 = _xmat(h0, Wcat0, N, D)
    agg0 = _edge_agg(X0.reshape((R + 1) * N, D),
                     fd3d, N, D, nch_e, stripe)
    X1 = _combine_xmat(agg0, X0, b0, Wcat1, N, D)
    agg1 = _edge_agg(X1.reshape((R + 1) * N, D),
                     fd3d, N, D, nch_e, stripe)
    P, Q = _final_pq(agg1, X1, b1, Wt, Wb, bm, N, D, OUT)
    score = _trip_score(P, Q, t0, t2, T, OUT)
    return score


# speedup vs baseline: 2.4758x; 1.0004x over previous
import functools

import jax
import jax.numpy as jnp
from jax import lax
from jax.experimental import pallas as pl
from jax.experimental.pallas import tpu as pltpu
from jax.experimental.pallas import tpu_sc as plsc

f32 = jnp.float32
i32 = jnp.int32

NC, NS, LANES = 2, 16, 16
NW = NC * NS


def _mesh():
    return plsc.VectorSubcoreMesh(core_axis_name="c", subcore_axis_name="s",
                                  num_cores=NC, num_subcores=NS)


def _wid():
    return lax.axis_index("s") * NC + lax.axis_index("c")


def _emb_gather(emb, nfp, NP, D):
    npw = NP // NW
    CH = 64
    nch = npw // CH

    def body(emb_hbm, idx_hbm, out_hbm, idx_v, rows_v, sem):
        wid = _wid()
        for k in range(nch):
            base = wid * npw + k * CH
            pltpu.sync_copy(idx_hbm.at[pl.ds(base, CH)], idx_v)
            pltpu.async_copy(emb_hbm.at[idx_v], rows_v, sem).wait()
            pltpu.sync_copy(rows_v, out_hbm.at[pl.ds(base, CH)])

    return pl.kernel(
        body,
        out_type=jax.ShapeDtypeStruct((NP, D), f32),
        mesh=_mesh(),
        scratch_types=[
            pltpu.VMEM((CH,), i32),
            pltpu.VMEM((CH, D), f32),
            pltpu.SemaphoreType.DMA,
        ],
    )(emb, nfp)


def _edge_agg(xflat, fd3d, N, D, nch_e, stripe):
    acc_rows = NS * stripe
    GRP = 4
    ngr = nch_e // GRP

    def body(x_hbm, fd_hbm, out_hbm, ib0, ib1, r0, r1, acc, g0, g1):
        ibs = (ib0, ib1)
        rbufs = (r0, r1)
        gsems = (g0, g1)
        c = lax.axis_index("c")
        s = lax.axis_index("s")
        wid = s * NC + c
        gbase = wid * ngr

        z = jnp.zeros((LANES,), f32)

        @pl.loop(0, 128)
        def _(i):
            for j in range(D // LANES):
                r0[i, pl.ds(j * LANES, LANES)] = z

        for k in range(stripe // 128):
            pltpu.sync_copy(r0, acc.at[pl.ds(s * stripe + k * 128, 128)])
        plsc.subcore_barrier()

        def issue(idx_row, b):
            pltpu.async_copy(x_hbm.at[idx_row], rbufs[b], gsems[b])

        def wait_gather(b):
            pltpu.make_async_copy(x_hbm.at[pl.ds(0, 128)], rbufs[b],
                                  gsems[b]).wait()

        def do_group(g, par, load_next, last):
            ib, ibn = ibs[par], ibs[1 - par]
            if load_next:
                pltpu.sync_copy(fd_hbm.at[gbase + g + 1], ibn)
            for j in range(GRP):
                b = j % 2
                wait_gather(b)
                pltpu.sync_copy(rbufs[b], acc.at[ib.at[GRP + j]], add=True)
                if j < 2:
                    issue(ib.at[j + 2], b)
                elif not last:
                    issue(ibn.at[j - 2], b)

        pltpu.sync_copy(fd_hbm.at[gbase], ib0)
        issue(ib0.at[0], 0)
        issue(ib0.at[1], 1)

        @pl.loop(0, ngr // 2 - 1)
        def _(t):
            do_group(2 * t, 0, True, False)
            do_group(2 * t + 1, 1, True, False)

        do_group(ngr - 2, 0, True, False)
        do_group(ngr - 1, 1, False, True)

        plsc.subcore_barrier()
        for k in range(stripe // 128):
            rr = s * stripe + k * 128
            pltpu.sync_copy(acc.at[pl.ds(rr, 128)], r0)
            pltpu.sync_copy(r0, out_hbm.at[c, pl.ds(rr, 128)])

    return pl.kernel(
        body,
        out_type=jax.ShapeDtypeStruct((NC, acc_rows, D), f32),
        mesh=_mesh(),
        scratch_types=[
            pltpu.VMEM((8, 128), i32),
            pltpu.VMEM((8, 128), i32),
            pltpu.VMEM((128, D), f32),
            pltpu.VMEM((128, D), f32),
            pltpu.VMEM_SHARED((acc_rows, D), f32),
            pltpu.SemaphoreType.DMA,
            pltpu.SemaphoreType.DMA,
        ],
    )(xflat, fd3d)


def _trip_score(P, Q, t0, t2, T, OUT):
    tpw = T // NW
    CH = 80
    nch = tpw // CH

    assert nch % 2 == 1

    def body(p_hbm, q_hbm, t0_hbm, t2_hbm, out_hbm, t0buf, t2buf,
             pb0, qb0, ob0, pb1, qb1, ob1, sp0, sq0, sp1, sq1):
        wid = _wid()
        pltpu.sync_copy(t0_hbm.at[pl.ds(wid * tpw, tpw)], t0buf)
        pltpu.sync_copy(t2_hbm.at[pl.ds(wid * tpw, tpw)], t2buf)
        sets = ((pb0, qb0, ob0, sp0, sq0), (pb1, qb1, ob1, sp1, sq1))

        def issue(ch, st):
            pb, qb, _, sp, sq = st
            pltpu.async_copy(p_hbm.at[t0buf.at[pl.ds(ch * CH, CH)]], pb, sp)
            pltpu.async_copy(q_hbm.at[t2buf.at[pl.ds(ch * CH, CH)]], qb, sq)

        def process(ch, st):
            pb, qb, ob, sp, sq = st
            pltpu.make_async_copy(p_hbm.at[pl.ds(0, CH)], pb, sp).wait()
            pltpu.make_async_copy(q_hbm.at[pl.ds(0, CH)], qb, sq).wait()
            for i in range(CH):
                ob[i, :] = pb[i, :] + qb[i, :]
            pltpu.sync_copy(ob, out_hbm.at[pl.ds(wid * tpw + ch * CH, CH)])

        issue(0, sets[0])

        @pl.loop(0, nch // 2)
        def _(g):
            issue(2 * g + 1, sets[1])
            process(2 * g, sets[0])
            issue(2 * g + 2, sets[0])
            process(2 * g + 1, sets[1])

        process(nch - 1, sets[0])

    return pl.kernel(
        body,
        out_type=jax.ShapeDtypeStruct((T, OUT), f32),
        mesh=_mesh(),
        compiler_params=pltpu.CompilerParams(use_tc_tiling_on_sc=False),
        scratch_types=[
            pltpu.VMEM((tpw,), i32),
            pltpu.VMEM((tpw,), i32),
            pltpu.VMEM((CH, OUT), f32),
            pltpu.VMEM((CH, OUT), f32),
            pltpu.VMEM((CH, OUT), f32),
            pltpu.VMEM((CH, OUT), f32),
            pltpu.VMEM((CH, OUT), f32),
            pltpu.VMEM((CH, OUT), f32),
            pltpu.SemaphoreType.DMA,
            pltpu.SemaphoreType.DMA,
            pltpu.SemaphoreType.DMA,
            pltpu.SemaphoreType.DMA,
        ],
    )(P, Q, t0, t2)


_BLK = 2000


def _xmat(h, Wcat, N, D):
    R1 = Wcat.shape[0]
    nb = N // _BLK

    def body(h_ref, w_ref, o_ref):
        o_ref[...] = jnp.dot(h_ref[...], w_ref[0], preferred_element_type=f32)

    return pl.pallas_call(
        body,
        grid=(nb, R1),
        in_specs=[
            pl.BlockSpec((_BLK, D), lambda j, r: (j, 0)),
            pl.BlockSpec((1, D, D), lambda j, r: (r, 0, 0)),
        ],
        out_specs=pl.BlockSpec((_BLK, D), lambda j, r: (r * nb + j, 0)),
        out_shape=jax.ShapeDtypeStruct((R1 * N, D), f32),
    )(h, Wcat)


def _combine_xmat(agg, Xprev, b, Wcat, N, D):
    R1 = Wcat.shape[0]
    R = R1 - 1
    nb = N // _BLK

    def body(agg_ref, sx_ref, b_ref, w_ref, o_ref, h_scr):
        @pl.when(pl.program_id(1) == 0)
        def _():
            h_scr[...] = (agg_ref[0] + agg_ref[1] + sx_ref[...] + b_ref[0])
        o_ref[...] = jnp.dot(h_scr[...], w_ref[0], preferred_element_type=f32)

    return pl.pallas_call(
        body,
        grid=(nb, R1),
        in_specs=[
            pl.BlockSpec((NC, _BLK, D), lambda j, r: (0, j, 0)),
            pl.BlockSpec((_BLK, D), lambda j, r: (R * nb + j, 0)),
            pl.BlockSpec((1, D), lambda j, r: (0, 0)),
            pl.BlockSpec((1, D, D), lambda j, r: (r, 0, 0)),
        ],
        out_specs=pl.BlockSpec((_BLK, D), lambda j, r: (r * nb + j, 0)),
        out_shape=jax.ShapeDtypeStruct((R1 * N, D), f32),
        scratch_shapes=[pltpu.VMEM((_BLK, D), f32)],
    )(agg, Xprev, b, Wcat)


def _final_pq(agg, Xprev, b, Wt, Wb, bm, N, D, OUT):
    R = Xprev.shape[0] // N - 1
    nb = N // _BLK

    def body(agg_ref, sx_ref, b_ref, wt_ref, wb_ref, bm_ref, p_ref, q_ref):
        h2 = agg_ref[0] + agg_ref[1] + sx_ref[...] + b_ref[0]
        p_ref[...] = jnp.dot(h2, wt_ref[...], preferred_element_type=f32) \
            + bm_ref[0]
        q_ref[...] = jnp.dot(h2, wb_ref[...], preferred_element_type=f32)

    return pl.pallas_call(
        body,
        grid=(nb,),
        in_specs=[
            pl.BlockSpec((NC, _BLK, D), lambda j: (0, j, 0)),
            pl.BlockSpec((_BLK, D), lambda j: (R * nb + j, 0)),
            pl.BlockSpec((1, D), lambda j: (0, 0)),
            pl.BlockSpec((D, OUT), lambda j: (0, 0)),
            pl.BlockSpec((D, OUT), lambda j: (0, 0)),
            pl.BlockSpec((1, OUT), lambda j: (0, 0)),
        ],
        out_specs=[
            pl.BlockSpec((_BLK, OUT), lambda j: (j, 0)),
            pl.BlockSpec((_BLK, OUT), lambda j: (j, 0)),
        ],
        out_shape=[
            jax.ShapeDtypeStruct((N, OUT), f32),
            jax.ShapeDtypeStruct((N, OUT), f32),
        ],
    )(agg, Xprev, b, Wt, Wb, bm)


def kernel(node_feat, edge_index, edge_feat, triplets, emb,
           W_rel_0, W_self_0, b_0, W_rel_1, W_self_1, b_1, W_mlp, b_mlp):
    N, D = emb.shape
    R = W_rel_0.shape[0]
    E = edge_feat.shape[0]
    T = triplets.shape[0]
    OUT = W_mlp.shape[1]

    src = edge_index[0].astype(i32)
    dst = edge_index[1].astype(i32)
    et = edge_feat.astype(i32)
    flat = et * N + src

    nch_e = 8 * (-(-E // (NW * 128 * 8)))
    EP = NW * nch_e * 128
    ngr = nch_e // 4
    stripe0 = 128 * (-(-(N + 1) // (NS * 128)))
    trash = N + jnp.arange(EP - E, dtype=i32) % (NS * stripe0 - N)
    flatp = jnp.concatenate([flat, jnp.zeros((EP - E,), i32)])
    dstp = jnp.concatenate([dst, trash])
    fd3d = jnp.concatenate(
        [flatp.reshape(NW, ngr, 4, 128), dstp.reshape(NW, ngr, 4, 128)],
        axis=2).reshape(NW * ngr, 8, 128)
    stripe = stripe0

    NP = NW * 64 * (-(-N // (NW * 64)))
    nfp = jnp.concatenate([node_feat.astype(i32),
                           jnp.zeros((NP - N,), i32)])

    t0 = triplets[:, 0].astype(i32)
    t2 = triplets[:, 2].astype(i32)

    Wcat0 = jnp.concatenate([W_rel_0, W_self_0[None]], axis=0)
    Wcat1 = jnp.concatenate([W_rel_1, W_self_1[None]], axis=0)
    b0 = b_0.reshape(1, D)
    b1 = b_1.reshape(1, D)
    bm = b_mlp.reshape(1, OUT)
    Wt = W_mlp[:D]
    Wb = W_mlp[D:]

    h0 = _emb_gather(emb, nfp, NP, D)
    X0 = _xmat(h0, Wcat0, N, D)
    agg0 = _edge_agg(X0, fd3d, N, D, nch_e, stripe)
    X1 = _combine_xmat(agg0, X0, b0, Wcat1, N, D)
    agg1 = _edge_agg(X1, fd3d, N, D, nch_e, stripe)
    P, Q = _final_pq(agg1, X1, b1, Wt, Wb, bm, N, D, OUT)
    score = _trip_score(P, Q, t0, t2, T, OUT)
    return score


# speedup vs baseline: 2.4760x; 1.0000x over previous
import functools

import jax
import jax.numpy as jnp
from jax import lax
from jax.experimental import pallas as pl
from jax.experimental.pallas import tpu as pltpu
from jax.experimental.pallas import tpu_sc as plsc

f32 = jnp.float32
i32 = jnp.int32

NC, NS, LANES = 2, 16, 16
NW = NC * NS


def _mesh():
    return plsc.VectorSubcoreMesh(core_axis_name="c", subcore_axis_name="s",
                                  num_cores=NC, num_subcores=NS)


def _wid():
    return lax.axis_index("s") * NC + lax.axis_index("c")


def _emb_gather(emb, nfp, NP, D):
    npw = NP // NW
    CH = 64
    nch = npw // CH

    def body(emb_hbm, idx_hbm, out_hbm, idx_v, rows_v, sem):
        wid = _wid()
        for k in range(nch):
            base = wid * npw + k * CH
            pltpu.sync_copy(idx_hbm.at[pl.ds(base, CH)], idx_v)
            pltpu.async_copy(emb_hbm.at[idx_v], rows_v, sem).wait()
            pltpu.sync_copy(rows_v, out_hbm.at[pl.ds(base, CH)])

    return pl.kernel(
        body,
        out_type=jax.ShapeDtypeStruct((NP, D), f32),
        mesh=_mesh(),
        compiler_params=pltpu.CompilerParams(use_tc_tiling_on_sc=False),
        scratch_types=[
            pltpu.VMEM((CH,), i32),
            pltpu.VMEM((CH, D), f32),
            pltpu.SemaphoreType.DMA,
        ],
    )(emb, nfp)


def _edge_agg(xflat, fd3d, N, D, nch_e, stripe):
    acc_rows = NS * stripe
    GRP = 4
    ngr = nch_e // GRP

    def body(x_hbm, fd_hbm, out_hbm, ib0, ib1, r0, r1, acc, g0, g1):
        ibs = (ib0, ib1)
        rbufs = (r0, r1)
        gsems = (g0, g1)
        c = lax.axis_index("c")
        s = lax.axis_index("s")
        wid = s * NC + c
        gbase = wid * ngr

        z = jnp.zeros((LANES,), f32)

        @pl.loop(0, 128)
        def _(i):
            for j in range(D // LANES):
                r0[i, pl.ds(j * LANES, LANES)] = z

        for k in range(stripe // 128):
            pltpu.sync_copy(r0, acc.at[pl.ds(s * stripe + k * 128, 128)])
        plsc.subcore_barrier()

        def issue(idx_row, b):
            pltpu.async_copy(x_hbm.at[idx_row], rbufs[b], gsems[b])

        def wait_gather(b):
            pltpu.make_async_copy(x_hbm.at[pl.ds(0, 128)], rbufs[b],
                                  gsems[b]).wait()

        def do_group(g, par, load_next, last):
            ib, ibn = ibs[par], ibs[1 - par]
            if load_next:
                pltpu.sync_copy(fd_hbm.at[gbase + g + 1], ibn)
            for j in range(GRP):
                b = j % 2
                wait_gather(b)
                pltpu.sync_copy(rbufs[b], acc.at[ib.at[GRP + j]], add=True)
                if j < 2:
                    issue(ib.at[j + 2], b)
                elif not last:
                    issue(ibn.at[j - 2], b)

        pltpu.sync_copy(fd_hbm.at[gbase], ib0)
        issue(ib0.at[0], 0)
        issue(ib0.at[1], 1)

        @pl.loop(0, ngr // 2 - 1)
        def _(t):
            do_group(2 * t, 0, True, False)
            do_group(2 * t + 1, 1, True, False)

        do_group(ngr - 2, 0, True, False)
        do_group(ngr - 1, 1, False, True)

        plsc.subcore_barrier()
        for k in range(stripe // 128):
            rr = s * stripe + k * 128
            pltpu.sync_copy(acc.at[pl.ds(rr, 128)], r0)
            pltpu.sync_copy(r0, out_hbm.at[c, pl.ds(rr, 128)])

    return pl.kernel(
        body,
        out_type=jax.ShapeDtypeStruct((NC, acc_rows, D), f32),
        mesh=_mesh(),
        compiler_params=pltpu.CompilerParams(use_tc_tiling_on_sc=False),
        scratch_types=[
            pltpu.VMEM((8, 128), i32),
            pltpu.VMEM((8, 128), i32),
            pltpu.VMEM((128, D), f32),
            pltpu.VMEM((128, D), f32),
            pltpu.VMEM_SHARED((acc_rows, D), f32),
            pltpu.SemaphoreType.DMA,
            pltpu.SemaphoreType.DMA,
        ],
    )(xflat, fd3d)


def _trip_score(P, Q, t0, t2, T, OUT):
    tpw = T // NW
    CH = 80
    nch = tpw // CH

    assert nch % 2 == 1

    def body(p_hbm, q_hbm, t0_hbm, t2_hbm, out_hbm, t0buf, t2buf,
             pb0, qb0, ob0, pb1, qb1, ob1, sp0, sq0, sp1, sq1):
        wid = _wid()
        pltpu.sync_copy(t0_hbm.at[pl.ds(wid * tpw, tpw)], t0buf)
        pltpu.sync_copy(t2_hbm.at[pl.ds(wid * tpw, tpw)], t2buf)
        sets = ((pb0, qb0, ob0, sp0, sq0), (pb1, qb1, ob1, sp1, sq1))

        def issue(ch, st):
            pb, qb, _, sp, sq = st
            pltpu.async_copy(p_hbm.at[t0buf.at[pl.ds(ch * CH, CH)]], pb, sp)
            pltpu.async_copy(q_hbm.at[t2buf.at[pl.ds(ch * CH, CH)]], qb, sq)

        def process(ch, st):
            pb, qb, ob, sp, sq = st
            pltpu.make_async_copy(p_hbm.at[pl.ds(0, CH)], pb, sp).wait()
            pltpu.make_async_copy(q_hbm.at[pl.ds(0, CH)], qb, sq).wait()
            for i in range(CH):
                ob[i, :] = pb[i, :] + qb[i, :]
            pltpu.sync_copy(ob, out_hbm.at[pl.ds(wid * tpw + ch * CH, CH)])

        issue(0, sets[0])

        @pl.loop(0, nch // 2)
        def _(g):
            issue(2 * g + 1, sets[1])
            process(2 * g, sets[0])
            issue(2 * g + 2, sets[0])
            process(2 * g + 1, sets[1])

        process(nch - 1, sets[0])

    return pl.kernel(
        body,
        out_type=jax.ShapeDtypeStruct((T, OUT), f32),
        mesh=_mesh(),
        compiler_params=pltpu.CompilerParams(use_tc_tiling_on_sc=False),
        scratch_types=[
            pltpu.VMEM((tpw,), i32),
            pltpu.VMEM((tpw,), i32),
            pltpu.VMEM((CH, OUT), f32),
            pltpu.VMEM((CH, OUT), f32),
            pltpu.VMEM((CH, OUT), f32),
            pltpu.VMEM((CH, OUT), f32),
            pltpu.VMEM((CH, OUT), f32),
            pltpu.VMEM((CH, OUT), f32),
            pltpu.SemaphoreType.DMA,
            pltpu.SemaphoreType.DMA,
            pltpu.SemaphoreType.DMA,
            pltpu.SemaphoreType.DMA,
        ],
    )(P, Q, t0, t2)


_BLK = 2000


def _xmat(h, Wcat, N, D):
    R1 = Wcat.shape[0]
    nb = N // _BLK

    def body(h_ref, w_ref, o_ref):
        o_ref[...] = jnp.dot(h_ref[...], w_ref[0], preferred_element_type=f32)

    return pl.pallas_call(
        body,
        grid=(nb, R1),
        in_specs=[
            pl.BlockSpec((_BLK, D), lambda j, r: (j, 0)),
            pl.BlockSpec((1, D, D), lambda j, r: (r, 0, 0)),
        ],
        out_specs=pl.BlockSpec((_BLK, D), lambda j, r: (r * nb + j, 0)),
        out_shape=jax.ShapeDtypeStruct((R1 * N, D), f32),
    )(h, Wcat)


def _combine_xmat(agg, Xprev, b, Wcat, N, D):
    R1 = Wcat.shape[0]
    R = R1 - 1
    nb = N // _BLK

    def body(agg_ref, sx_ref, b_ref, w_ref, o_ref, h_scr):
        @pl.when(pl.program_id(1) == 0)
        def _():
            h_scr[...] = (agg_ref[0] + agg_ref[1] + sx_ref[...] + b_ref[0])
        o_ref[...] = jnp.dot(h_scr[...], w_ref[0], preferred_element_type=f32)

    return pl.pallas_call(
        body,
        grid=(nb, R1),
        in_specs=[
            pl.BlockSpec((NC, _BLK, D), lambda j, r: (0, j, 0)),
            pl.BlockSpec((_BLK, D), lambda j, r: (R * nb + j, 0)),
            pl.BlockSpec((1, D), lambda j, r: (0, 0)),
            pl.BlockSpec((1, D, D), lambda j, r: (r, 0, 0)),
        ],
        out_specs=pl.BlockSpec((_BLK, D), lambda j, r: (r * nb + j, 0)),
        out_shape=jax.ShapeDtypeStruct((R1 * N, D), f32),
        scratch_shapes=[pltpu.VMEM((_BLK, D), f32)],
    )(agg, Xprev, b, Wcat)


def _final_pq(agg, Xprev, b, Wt, Wb, bm, N, D, OUT):
    R = Xprev.shape[0] // N - 1
    nb = N // _BLK

    def body(agg_ref, sx_ref, b_ref, wt_ref, wb_ref, bm_ref, p_ref, q_ref):
        h2 = agg_ref[0] + agg_ref[1] + sx_ref[...] + b_ref[0]
        p_ref[...] = jnp.dot(h2, wt_ref[...], preferred_element_type=f32) \
            + bm_ref[0]
        q_ref[...] = jnp.dot(h2, wb_ref[...], preferred_element_type=f32)

    return pl.pallas_call(
        body,
        grid=(nb,),
        in_specs=[
            pl.BlockSpec((NC, _BLK, D), lambda j: (0, j, 0)),
            pl.BlockSpec((_BLK, D), lambda j: (R * nb + j, 0)),
            pl.BlockSpec((1, D), lambda j: (0, 0)),
            pl.BlockSpec((D, OUT), lambda j: (0, 0)),
            pl.BlockSpec((D, OUT), lambda j: (0, 0)),
            pl.BlockSpec((1, OUT), lambda j: (0, 0)),
        ],
        out_specs=[
            pl.BlockSpec((_BLK, OUT), lambda j: (j, 0)),
            pl.BlockSpec((_BLK, OUT), lambda j: (j, 0)),
        ],
        out_shape=[
            jax.ShapeDtypeStruct((N, OUT), f32),
            jax.ShapeDtypeStruct((N, OUT), f32),
        ],
    )(agg, Xprev, b, Wt, Wb, bm)


def kernel(node_feat, edge_index, edge_feat, triplets, emb,
           W_rel_0, W_self_0, b_0, W_rel_1, W_self_1, b_1, W_mlp, b_mlp):
    N, D = emb.shape
    R = W_rel_0.shape[0]
    E = edge_feat.shape[0]
    T = triplets.shape[0]
    OUT = W_mlp.shape[1]

    src = edge_index[0].astype(i32)
    dst = edge_index[1].astype(i32)
    et = edge_feat.astype(i32)
    flat = et * N + src

    nch_e = 8 * (-(-E // (NW * 128 * 8)))
    EP = NW * nch_e * 128
    ngr = nch_e // 4
    stripe0 = 128 * (-(-(N + 1) // (NS * 128)))
    trash = N + jnp.arange(EP - E, dtype=i32) % (NS * stripe0 - N)
    flatp = jnp.concatenate([flat, jnp.zeros((EP - E,), i32)])
    dstp = jnp.concatenate([dst, trash])
    fd3d = jnp.concatenate(
        [flatp.reshape(NW, ngr, 4, 128), dstp.reshape(NW, ngr, 4, 128)],
        axis=2).reshape(NW * ngr, 8, 128)
    stripe = stripe0

    NP = NW * 64 * (-(-N // (NW * 64)))
    nfp = jnp.concatenate([node_feat.astype(i32),
                           jnp.zeros((NP - N,), i32)])

    t0 = triplets[:, 0].astype(i32)
    t2 = triplets[:, 2].astype(i32)

    Wcat0 = jnp.concatenate([W_rel_0, W_self_0[None]], axis=0)
    Wcat1 = jnp.concatenate([W_rel_1, W_self_1[None]], axis=0)
    b0 = b_0.reshape(1, D)
    b1 = b_1.reshape(1, D)
    bm = b_mlp.reshape(1, OUT)
    Wt = W_mlp[:D]
    Wb = W_mlp[D:]

    h0 = _emb_gather(emb, nfp, NP, D)
    X0 = _xmat(h0, Wcat0, N, D)
    agg0 = _edge_agg(X0, fd3d, N, D, nch_e, stripe)
    X1 = _combine_xmat(agg0, X0, b0, Wcat1, N, D)
    agg1 = _edge_agg(X1, fd3d, N, D, nch_e, stripe)
    P, Q = _final_pq(agg1, X1, b1, Wt, Wb, bm, N, D, OUT)
    score = _trip_score(P, Q, t0, t2, T, OUT)
    return score
